# baseline (speedup 1.0000x reference)
import functools

import jax
import jax.numpy as jnp
from jax import lax
from jax.experimental import pallas as pl
from jax.experimental.pallas import tpu as pltpu
from jax.experimental.pallas import tpu_sc as plsc

B = 16384
D = 64
CH = 128

_info = plsc.get_sparse_core_info()
NC, NS, L = _info.num_cores, _info.num_subcores, _info.num_lanes
NW = NC * NS
BPW = B // NW
NCH = BPW // CH
GPC = CH // L


def _sc_body(subj_hbm, obj_hbm, obs_hbm, samp_hbm, arg_hbm, rel_hbm,
             pos_hbm, neg_hbm,
             sidx, oidx, obsidx, sampidx,
             srows, orows, obsrows, samprows,
             posbuf, negbuf, sem):
    wid = lax.axis_index("s") * NC + lax.axis_index("c")

    pltpu.sync_copy(subj_hbm.at[wid], sidx)
    pltpu.sync_copy(obj_hbm.at[wid], oidx)
    pltpu.sync_copy(obs_hbm.at[wid], obsidx)
    pltpu.sync_copy(samp_hbm.at[wid], sampidx)

    lanes = lax.iota(jnp.int32, L)

    for j in range(NCH):
        c1 = pltpu.async_copy(arg_hbm.at[sidx.at[j]], srows, sem)
        c2 = pltpu.async_copy(arg_hbm.at[oidx.at[j]], orows, sem)
        c3 = pltpu.async_copy(rel_hbm.at[obsidx.at[j]], obsrows, sem)
        c4 = pltpu.async_copy(rel_hbm.at[sampidx.at[j]], samprows, sem)
        c1.wait()
        c2.wait()
        c3.wait()
        c4.wait()

        def group_body(g, _, j=j):
            rv = g * L + lanes

            def d_body(d, acc):
                accp, accn = acc
                dv = jnp.full((L,), 0, jnp.int32) + d
                s = plsc.load_gather(srows, [rv, dv])
                o = plsc.load_gather(orows, [rv, dv])
                ob = plsc.load_gather(obsrows, [rv, dv])
                sa = plsc.load_gather(samprows, [rv, dv])
                p = s * o
                return accp + p * ob, accn + p * sa

            zero = jnp.zeros((L,), jnp.float32)
            accp, accn = lax.fori_loop(0, D, d_body, (zero, zero),
                                       unroll=8)
            posbuf[j * GPC + g] = accp
            negbuf[j * GPC + g] = accn
            return 0

        lax.fori_loop(0, GPC, group_body, 0)

    pltpu.sync_copy(posbuf, pos_hbm.at[wid])
    pltpu.sync_copy(negbuf, neg_hbm.at[wid])


_sc_call = functools.partial(
    pl.kernel,
    mesh=plsc.VectorSubcoreMesh(core_axis_name="c", subcore_axis_name="s"),
    compiler_params=pltpu.CompilerParams(needs_layout_passes=False,
                                         use_tc_tiling_on_sc=False),
    out_type=(jax.ShapeDtypeStruct((NW, BPW // L, L), jnp.float32),
              jax.ShapeDtypeStruct((NW, BPW // L, L), jnp.float32)),
    scratch_types=[
        pltpu.VMEM((NCH, CH), jnp.int32),
        pltpu.VMEM((NCH, CH), jnp.int32),
        pltpu.VMEM((NCH, CH), jnp.int32),
        pltpu.VMEM((NCH, CH), jnp.int32),
        pltpu.VMEM((CH, D), jnp.float32),
        pltpu.VMEM((CH, D), jnp.float32),
        pltpu.VMEM((CH, D), jnp.float32),
        pltpu.VMEM((CH, D), jnp.float32),
        pltpu.VMEM((BPW // L, L), jnp.float32),
        pltpu.VMEM((BPW // L, L), jnp.float32),
        pltpu.SemaphoreType.DMA,
    ],
)(_sc_body)


def kernel(subjects, objects, observed_relations, sampled_relations,
           arg_table, rel_table):
    subj = subjects.astype(jnp.int32).reshape(NW, NCH, CH)
    obj = objects.astype(jnp.int32).reshape(NW, NCH, CH)
    obs = observed_relations.astype(jnp.int32).reshape(NW, NCH, CH)
    samp = sampled_relations.astype(jnp.int32).reshape(NW, NCH, CH)
    pos, neg = _sc_call(subj, obj, obs, samp, arg_table, rel_table)
    return pos.reshape(B), neg.reshape(B)


# speedup vs baseline: 1.3350x; 1.3350x over previous
import functools

import jax
import jax.numpy as jnp
from jax import lax
from jax.experimental import pallas as pl
from jax.experimental.pallas import tpu as pltpu
from jax.experimental.pallas import tpu_sc as plsc

B = 16384
D = 64
CH = 128

_info = plsc.get_sparse_core_info()
NC, NS, L = _info.num_cores, _info.num_subcores, _info.num_lanes
NW = NC * NS
BPW = B // NW
NCH = BPW // CH
GPC = CH // L


def _sc_body(subj_hbm, obj_hbm, obs_hbm, samp_hbm, arg_hbm, rel_hbm,
             pos_hbm, neg_hbm,
             sidx, oidx, obsidx, sampidx,
             srows, orows, obsrows, samprows,
             posbuf, negbuf, sem):
    wid = lax.axis_index("s") * NC + lax.axis_index("c")
    base = wid * BPW

    pltpu.sync_copy(subj_hbm.at[wid], sidx)
    pltpu.sync_copy(obj_hbm.at[wid], oidx)
    pltpu.sync_copy(obs_hbm.at[wid], obsidx)
    pltpu.sync_copy(samp_hbm.at[wid], sampidx)

    lanes = lax.iota(jnp.int32, L)

    def fetch_rows(table, idx_ref, j, rows):
        def g_body(g, _):
            iv = idx_ref[j, pl.ds(g * L, L)]
            copies = [
                pltpu.async_copy(table.at[pl.ds(iv[k], 1)],
                                 rows.at[pl.ds(g * L + k, 1)], sem)
                for k in range(L)
            ]
            for c in copies:
                c.wait()
            return 0

        lax.fori_loop(0, GPC, g_body, 0)

    for j in range(NCH):
        fetch_rows(arg_hbm, sidx, j, srows)
        fetch_rows(arg_hbm, oidx, j, orows)
        fetch_rows(rel_hbm, obsidx, j, obsrows)
        fetch_rows(rel_hbm, sampidx, j, samprows)

        def group_body(g, _, j=j):
            rv = g * L + lanes

            def d_body(d, acc):
                accp, accn = acc
                dv = jnp.full((L,), 0, jnp.int32) + d
                s = plsc.load_gather(srows, [rv, dv])
                o = plsc.load_gather(orows, [rv, dv])
                ob = plsc.load_gather(obsrows, [rv, dv])
                sa = plsc.load_gather(samprows, [rv, dv])
                p = s * o
                return accp + p * ob, accn + p * sa

            zero = jnp.zeros((L,), jnp.float32)
            accp, accn = lax.fori_loop(0, D, d_body, (zero, zero),
                                       unroll=8)
            posbuf[pl.ds((j * GPC + g) * L, L)] = accp
            negbuf[pl.ds((j * GPC + g) * L, L)] = accn
            return 0

        lax.fori_loop(0, GPC, group_body, 0)

    pltpu.sync_copy(posbuf, pos_hbm.at[pl.ds(base, BPW)])
    pltpu.sync_copy(negbuf, neg_hbm.at[pl.ds(base, BPW)])


_sc_call = functools.partial(
    pl.kernel,
    mesh=plsc.VectorSubcoreMesh(core_axis_name="c", subcore_axis_name="s"),
    compiler_params=pltpu.CompilerParams(needs_layout_passes=False),
    out_type=(jax.ShapeDtypeStruct((B,), jnp.float32),
              jax.ShapeDtypeStruct((B,), jnp.float32)),
    scratch_types=[
        pltpu.VMEM((NCH, CH), jnp.int32),
        pltpu.VMEM((NCH, CH), jnp.int32),
        pltpu.VMEM((NCH, CH), jnp.int32),
        pltpu.VMEM((NCH, CH), jnp.int32),
        pltpu.VMEM((CH, D), jnp.float32),
        pltpu.VMEM((CH, D), jnp.float32),
        pltpu.VMEM((CH, D), jnp.float32),
        pltpu.VMEM((CH, D), jnp.float32),
        pltpu.VMEM((BPW,), jnp.float32),
        pltpu.VMEM((BPW,), jnp.float32),
        pltpu.SemaphoreType.DMA,
    ],
)(_sc_body)


def kernel(subjects, objects, observed_relations, sampled_relations,
           arg_table, rel_table):
    subj = subjects.astype(jnp.int32).reshape(NW, NCH, CH)
    obj = objects.astype(jnp.int32).reshape(NW, NCH, CH)
    obs = observed_relations.astype(jnp.int32).reshape(NW, NCH, CH)
    samp = sampled_relations.astype(jnp.int32).reshape(NW, NCH, CH)
    pos, neg = _sc_call(subj, obj, obs, samp, arg_table, rel_table)
    return pos, neg


# speedup vs baseline: 1.5421x; 1.1551x over previous
import functools

import jax
import jax.numpy as jnp
from jax import lax
from jax.experimental import pallas as pl
from jax.experimental.pallas import tpu as pltpu
from jax.experimental.pallas import tpu_sc as plsc

B = 16384
D = 64
CH = 128

_info = plsc.get_sparse_core_info()
NC, NS, L = _info.num_cores, _info.num_subcores, _info.num_lanes
NW = NC * NS
BPW = B // NW
NCH = BPW // CH
GPC = CH // L


def _sc_body(subj_hbm, obj_hbm, obs_hbm, samp_hbm, arg_hbm, rel_hbm,
             pos_hbm, neg_hbm,
             sidx, oidx, obsidx, sampidx,
             srows, orows, obsrows, samprows,
             posbuf, negbuf, sem):
    wid = lax.axis_index("s") * NC + lax.axis_index("c")
    base = wid * BPW

    pltpu.sync_copy(subj_hbm.at[wid], sidx)
    pltpu.sync_copy(obj_hbm.at[wid], oidx)
    pltpu.sync_copy(obs_hbm.at[wid], obsidx)
    pltpu.sync_copy(samp_hbm.at[wid], sampidx)

    lanes = lax.iota(jnp.int32, L)

    def fire_rows(table, idx_ref, j, rows):
        def g_body(g, _):
            iv = idx_ref[j, pl.ds(g * L, L)]
            for k in range(L):
                pltpu.async_copy(table.at[pl.ds(iv[k], 1)],
                                 rows.at[pl.ds(g * L + k, 1)], sem)
            return 0

        lax.fori_loop(0, GPC, g_body, 0)

    def drain(table, rows):
        pltpu.make_async_copy(table.at[pl.ds(0, CH)], rows, sem).wait()

    for j in range(NCH):
        fire_rows(arg_hbm, sidx, j, srows)
        fire_rows(arg_hbm, oidx, j, orows)
        fire_rows(rel_hbm, obsidx, j, obsrows)
        fire_rows(rel_hbm, sampidx, j, samprows)
        drain(arg_hbm, srows)
        drain(arg_hbm, orows)
        drain(rel_hbm, obsrows)
        drain(rel_hbm, samprows)

        def group_body(g, _, j=j):
            rv = g * L + lanes

            def d_body(d, acc):
                accp, accn = acc
                dv = jnp.full((L,), 0, jnp.int32) + d
                s = plsc.load_gather(srows, [rv, dv])
                o = plsc.load_gather(orows, [rv, dv])
                ob = plsc.load_gather(obsrows, [rv, dv])
                sa = plsc.load_gather(samprows, [rv, dv])
                p = s * o
                return accp + p * ob, accn + p * sa

            zero = jnp.zeros((L,), jnp.float32)
            accp, accn = lax.fori_loop(0, D, d_body, (zero, zero),
                                       unroll=8)
            posbuf[pl.ds((j * GPC + g) * L, L)] = accp
            negbuf[pl.ds((j * GPC + g) * L, L)] = accn
            return 0

        lax.fori_loop(0, GPC, group_body, 0)

    pltpu.sync_copy(posbuf, pos_hbm.at[pl.ds(base, BPW)])
    pltpu.sync_copy(negbuf, neg_hbm.at[pl.ds(base, BPW)])


_sc_call = functools.partial(
    pl.kernel,
    mesh=plsc.VectorSubcoreMesh(core_axis_name="c", subcore_axis_name="s"),
    compiler_params=pltpu.CompilerParams(needs_layout_passes=False),
    out_type=(jax.ShapeDtypeStruct((B,), jnp.float32),
              jax.ShapeDtypeStruct((B,), jnp.float32)),
    scratch_types=[
        pltpu.VMEM((NCH, CH), jnp.int32),
        pltpu.VMEM((NCH, CH), jnp.int32),
        pltpu.VMEM((NCH, CH), jnp.int32),
        pltpu.VMEM((NCH, CH), jnp.int32),
        pltpu.VMEM((CH, D), jnp.float32),
        pltpu.VMEM((CH, D), jnp.float32),
        pltpu.VMEM((CH, D), jnp.float32),
        pltpu.VMEM((CH, D), jnp.float32),
        pltpu.VMEM((BPW,), jnp.float32),
        pltpu.VMEM((BPW,), jnp.float32),
        pltpu.SemaphoreType.DMA,
    ],
)(_sc_body)


def kernel(subjects, objects, observed_relations, sampled_relations,
           arg_table, rel_table):
    subj = subjects.astype(jnp.int32).reshape(NW, NCH, CH)
    obj = objects.astype(jnp.int32).reshape(NW, NCH, CH)
    obs = observed_relations.astype(jnp.int32).reshape(NW, NCH, CH)
    samp = sampled_relations.astype(jnp.int32).reshape(NW, NCH, CH)
    pos, neg = _sc_call(subj, obj, obs, samp, arg_table, rel_table)
    return pos, neg
